# baseline (speedup 1.0000x reference)
import functools
import jax
import jax.numpy as jnp
from jax import lax
from jax.experimental import pallas as pl
from jax.experimental.pallas import tpu as pltpu
from jax.experimental.pallas import tpu_sc as plsc

B = 1024
H = 20
E = 8
CE = 64
PE = 64
OFFS = 64
NCLUST = 4096
NCAND = 4
DPFH = 3
BH = B * H
NW = 32
PC_PER = B // NW
BH_PER = BH // NW
GRPS = BH_PER // 16

_mesh = plsc.VectorSubcoreMesh(core_axis_name="c", subcore_axis_name="s")
_f32 = jnp.float32
_sc_params = pltpu.CompilerParams(use_tc_tiling_on_sc=False,
                                  needs_layout_passes=False)


@functools.partial(
    pl.kernel,
    out_type=[
        jax.ShapeDtypeStruct((B, PE), _f32),
        jax.ShapeDtypeStruct((BH, CE), _f32),
    ],
    mesh=_mesh,
    scratch_types=[
        pltpu.VMEM((PC_PER,), jnp.int32),
        pltpu.VMEM((PC_PER, PE), _f32),
        pltpu.VMEM((BH_PER,), jnp.int32),
        pltpu.VMEM((BH_PER, CE), _f32),
        pltpu.SemaphoreType.DMA,
    ],
    compiler_params=_sc_params,
)
def _gather_embeds(pc_idx, ch_idx, pc_table, cluster_table,
                   pc_out, cl_out, pidx_v, prow_v, cidx_v, crow_v, sem):
    wid = lax.axis_index("s") * 2 + lax.axis_index("c")
    pbase = wid * PC_PER
    cbase = wid * BH_PER
    pltpu.sync_copy(pc_idx.at[pl.ds(pbase, PC_PER)], pidx_v)
    cp = pltpu.async_copy(pc_table.at[pidx_v], prow_v, sem)
    pltpu.sync_copy(ch_idx.at[pl.ds(cbase, BH_PER)], cidx_v)
    cc = pltpu.async_copy(cluster_table.at[cidx_v], crow_v, sem)
    cp.wait()
    pltpu.sync_copy(prow_v, pc_out.at[pl.ds(pbase, PC_PER)])
    cc.wait()
    pltpu.sync_copy(crow_v, cl_out.at[pl.ds(cbase, BH_PER)])


@functools.partial(
    pl.kernel,
    out_type=jax.ShapeDtypeStruct((BH, CE), _f32),
    mesh=_mesh,
    scratch_types=[
        pltpu.VMEM((BH_PER,), jnp.int32),
        pltpu.VMEM((BH_PER,), jnp.int32),
        pltpu.VMEM((BH_PER,), jnp.int32),
        pltpu.VMEM((BH_PER, E), _f32),
        pltpu.VMEM((BH_PER, E), _f32),
        pltpu.VMEM((OFFS * E * CE,), _f32),
        pltpu.VMEM((BH_PER, CE), _f32),
        pltpu.VMEM((BH_PER, CE), _f32),
        pltpu.SemaphoreType.DMA,
    ],
    compiler_params=_sc_params,
)
def _attn_ctx(st_rows, s1_rows, vwo_flat, idx0_hbm, idx1_hbm, off_hbm,
              jdst_hbm, ctx_out, idx0_v, idx1_v, off_v, s0_v, s1_v, vwo_v,
              out_v, out2_v, sem):
    wid = lax.axis_index("s") * 2 + lax.axis_index("c")
    base = wid * BH_PER
    pltpu.sync_copy(idx0_hbm.at[pl.ds(base, BH_PER)], idx0_v)
    c0 = pltpu.async_copy(st_rows.at[idx0_v], s0_v, sem)
    pltpu.sync_copy(idx1_hbm.at[pl.ds(base, BH_PER)], idx1_v)
    c1 = pltpu.async_copy(s1_rows.at[idx1_v], s1_v, sem)
    pltpu.sync_copy(off_hbm.at[pl.ds(base, BH_PER)], off_v)
    pltpu.sync_copy(vwo_flat, vwo_v)
    c0.wait()
    c1.wait()

    lanes = lax.iota(jnp.int32, 16)

    def group(g, carry):
        rows = g * 16 + lanes
        s0 = [plsc.load_gather(s0_v, [rows, jnp.full((16,), e, jnp.int32)])
              for e in range(E)]
        s1 = [plsc.load_gather(s1_v, [rows, jnp.full((16,), e, jnp.int32)])
              for e in range(E)]
        m0 = s0[0]
        m1 = s1[0]
        for e in range(1, E):
            m0 = jnp.maximum(m0, s0[e])
            m1 = jnp.maximum(m1, s1[e])
        p0 = [jnp.exp(x - m0) for x in s0]
        p1 = [jnp.exp(x - m1) for x in s1]
        z0 = p0[0]
        z1 = p1[0]
        for e in range(1, E):
            z0 = z0 + p0[e]
            z1 = z1 + p1[e]
        r0 = 0.5 / z0
        r1 = 0.5 / z1
        w = [p0[e] * r0 + p1[e] * r1 for e in range(E)]

        offv = off_v[pl.ds(g * 16, 16)]
        vbase = offv * (E * CE)
        off9 = offv * 9

        @plsc.parallel_loop(0, CE, unroll=4)
        def ctx_loop(c):
            cvec = jnp.broadcast_to(c, (16,))
            bsw = cvec + off9
            acc = jnp.zeros((16,), _f32)
            for e in range(E):
                idx = (vbase + e * CE) | ((bsw + e) & (CE - 1))
                acc = acc + w[e] * plsc.load_gather(vwo_v, [idx])
            plsc.store_scatter(out_v, [rows, (cvec + rows) & (CE - 1)], acc)

        return carry

    lax.fori_loop(0, GRPS, group, 0)

    @plsc.parallel_loop(0, BH_PER * CE // 16, unroll=4)
    def repack(i):
        j = i * 16 + lanes
        r = j >> 6
        c = j & (CE - 1)
        v = plsc.load_gather(out_v, [r, (c + r) & (CE - 1)])
        plsc.store_scatter(out2_v, [r, c], v)

    pltpu.sync_copy(jdst_hbm.at[pl.ds(base, BH_PER)], idx0_v)
    pltpu.async_copy(out2_v, ctx_out.at[idx0_v], sem).wait()


def _b2_body(ct_ref, wq_ref, ot2_ref, wk_ref, wv_ref, wo_ref,
             st_ref, vwo_ref, m1_ref):
    q = jnp.dot(ct_ref[...], wq_ref[...], preferred_element_type=_f32)
    k2 = jnp.dot(ot2_ref[...], wk_ref[...], preferred_element_type=_f32)
    st = lax.dot_general(q, k2, (((1,), (1,)), ((), ())),
                         preferred_element_type=_f32) * 0.125
    for t in range(4):
        st_ref[:, 8 * t:8 * t + 8, :] = \
            st[:, 128 * t:128 * t + 128].reshape(NCLUST // 64, 8, 128)

    @pl.when(pl.program_id(0) == 0)
    def _():
        v2 = jnp.dot(ot2_ref[...], wv_ref[...], preferred_element_type=_f32)
        vwo_ref[...] = jnp.dot(v2, wo_ref[...], preferred_element_type=_f32)
        m1 = lax.dot_general(wq_ref[...], k2, (((1,), (1,)), ((), ())),
                             preferred_element_type=_f32)
        m1_ref[...] = m1 * 0.125


def _b3_body(pce_ref, m1_ref, s1_ref):
    s1 = jnp.dot(pce_ref[...], m1_ref[...], preferred_element_type=_f32)
    for t in range(4):
        s1_ref[:, 8 * t:8 * t + 8, :] = \
            s1[:, 128 * t:128 * t + 128].reshape(B // 8, 8, 128)


def _d_body(pce_ref, cl_ref, ctx_ref, dpf_ref, wp_ref, wc_ref, wx_ref,
            wd_ref, bias_ref, out_ref):
    acc = jnp.dot(pce_ref[...], wp_ref[...], preferred_element_type=_f32)
    acc = acc + jnp.dot(dpf_ref[...], wd_ref[...], preferred_element_type=_f32)
    for t in range(H // 2):
        cl_t = cl_ref[:, t, :, :].reshape(B // 8, 128)
        ctx_t = ctx_ref[:, t, :, :].reshape(B // 8, 128)
        acc = acc + jnp.dot(cl_t, wc_ref[128 * t:128 * t + 128, :],
                            preferred_element_type=_f32)
        acc = acc + jnp.dot(ctx_t, wx_ref[128 * t:128 * t + 128, :],
                            preferred_element_type=_f32)
    out_ref[...] = acc + bias_ref[...]


def kernel(cluster_history, offset_history, pc, dpf_vectors, pc_table,
           cluster_table, offset_table, Wq, Wk, Wv, Wo, W_cand, b_cand,
           W_off, b_off):
    ch = cluster_history.reshape(-1)
    off = offset_history.reshape(-1)
    pc_idx = pc.reshape(-1)
    ot2 = offset_table.reshape(OFFS * E, CE)

    bh = lax.iota(jnp.int32, BH)
    bidx = bh // H
    hidx = bh % H
    jg = ((bidx >> 3) * 160 + (hidx >> 1) * 16 + ((bidx & 7) << 1) + (hidx & 1))
    jv = jg
    b2 = (bh // 160) * 8 + (bh % 16) // 2
    h2 = ((bh % 160) // 16) * 2 + (bh & 1)
    inv = b2 * H + h2
    ch_sc = ch[inv]

    pc_embed, cl_embed = _gather_embeds(pc_idx, ch_sc, pc_table, cluster_table)

    st_t, vwo, m1 = pl.pallas_call(
        _b2_body,
        grid=(8,),
        in_specs=[
            pl.BlockSpec((NCLUST // 8, CE), lambda i: (i, 0)),
            pl.BlockSpec((CE, CE), lambda i: (0, 0)),
            pl.BlockSpec((OFFS * E, CE), lambda i: (0, 0)),
            pl.BlockSpec((CE, CE), lambda i: (0, 0)),
            pl.BlockSpec((CE, CE), lambda i: (0, 0)),
            pl.BlockSpec((CE, CE), lambda i: (0, 0)),
        ],
        out_specs=[
            pl.BlockSpec((NCLUST // 64, 32, 128), lambda i: (i, 0, 0)),
            pl.BlockSpec((OFFS * E, CE), lambda i: (0, 0)),
            pl.BlockSpec((CE, OFFS * E), lambda i: (0, 0)),
        ],
        out_shape=[
            jax.ShapeDtypeStruct((NCLUST // 8, 32, 128), _f32),
            jax.ShapeDtypeStruct((OFFS * E, CE), _f32),
            jax.ShapeDtypeStruct((CE, OFFS * E), _f32),
        ],
    )(cluster_table, Wq, ot2, Wk, Wv, Wo)

    s1_t = pl.pallas_call(
        _b3_body,
        out_shape=jax.ShapeDtypeStruct((B // 8, 32, 128), _f32),
    )(pc_embed, m1)

    bidx = lax.iota(jnp.int32, BH) // H
    idx0 = ((ch >> 3) << 9) + ((off >> 4) << 7) + ((ch & 7) << 4) + (off & 15)
    idx1 = ((bidx >> 3) << 9) + ((off >> 4) << 7) + ((bidx & 7) << 4) + (off & 15)
    rr = lax.iota(jnp.int32, OFFS * E)[:, None]
    cc = lax.iota(jnp.int32, CE)[None, :]
    vwo_sw = jnp.take_along_axis(
        vwo, (cc - rr - (rr >> 3)) & (CE - 1), axis=1)
    ctx = _attn_ctx(st_t.reshape(NCLUST * OFFS, E), s1_t.reshape(B * OFFS, E),
                    vwo_sw.reshape(-1), idx0, idx1, off, jv)

    wfull = jnp.concatenate([W_cand, W_off], axis=1)
    bias = jnp.concatenate([b_cand, b_off]).reshape(1, NCAND + 1 + OFFS)
    nout = NCAND + 1 + OFFS
    out = pl.pallas_call(
        _d_body,
        grid=(8,),
        in_specs=[
            pl.BlockSpec((B // 8, PE), lambda i: (i, 0)),
            pl.BlockSpec((16, H // 2, 8, 128), lambda i: (i, 0, 0, 0)),
            pl.BlockSpec((16, H // 2, 8, 128), lambda i: (i, 0, 0, 0)),
            pl.BlockSpec((B // 8, DPFH * NCAND), lambda i: (i, 0)),
            pl.BlockSpec((PE, nout), lambda i: (0, 0)),
            pl.BlockSpec((H * CE, nout), lambda i: (0, 0)),
            pl.BlockSpec((H * CE, nout), lambda i: (0, 0)),
            pl.BlockSpec((DPFH * NCAND, nout), lambda i: (0, 0)),
            pl.BlockSpec((1, nout), lambda i: (0, 0)),
        ],
        out_specs=pl.BlockSpec((B // 8, nout), lambda i: (i, 0)),
        out_shape=jax.ShapeDtypeStruct((B, nout), _f32),
    )(pc_embed, cl_embed.reshape(128, H // 2, 8, 128),
      ctx.reshape(128, H // 2, 8, 128),
      dpf_vectors.reshape(B, DPFH * NCAND), wfull[:PE],
      wfull[PE:PE + H * CE], wfull[PE + H * CE:PE + 2 * H * CE],
      wfull[PE + 2 * H * CE:], bias)

    return (out[:, :NCAND + 1], out[:, NCAND + 1:])


# speedup vs baseline: 1.0258x; 1.0258x over previous
import functools
import jax
import jax.numpy as jnp
from jax import lax
from jax.experimental import pallas as pl
from jax.experimental.pallas import tpu as pltpu
from jax.experimental.pallas import tpu_sc as plsc

B = 1024
H = 20
E = 8
CE = 64
PE = 64
OFFS = 64
NCLUST = 4096
NCAND = 4
DPFH = 3
BH = B * H
NW = 32
PC_PER = B // NW
BH_PER = BH // NW
GRPS = BH_PER // 16

_mesh = plsc.VectorSubcoreMesh(core_axis_name="c", subcore_axis_name="s")
_f32 = jnp.float32
_sc_params = pltpu.CompilerParams(use_tc_tiling_on_sc=False,
                                  needs_layout_passes=False)


@functools.partial(
    pl.kernel,
    out_type=[
        jax.ShapeDtypeStruct((B, PE), _f32),
        jax.ShapeDtypeStruct((BH // 16, 8, 128), _f32),
    ],
    mesh=_mesh,
    scratch_types=[
        pltpu.VMEM((PC_PER,), jnp.int32),
        pltpu.VMEM((PC_PER, PE), _f32),
        pltpu.VMEM((BH_PER,), jnp.int32),
        pltpu.VMEM((BH_PER,), jnp.int32),
        pltpu.VMEM((BH_PER, CE), _f32),
        pltpu.VMEM((BH_PER // 16, 8, 128), _f32),
        pltpu.SemaphoreType.DMA,
    ],
    compiler_params=_sc_params,
)
def _gather_embeds(pc_idx, ch_idx, pc_table, cluster_table,
                   pc_out, cl_out, pidx_v, prow_v, cidx_v, cidx2_v, crow_v,
                   crow2_v, sem):
    wid = lax.axis_index("s") * 2 + lax.axis_index("c")
    pbase = wid * PC_PER
    cbase = wid * BH_PER
    pltpu.sync_copy(pc_idx.at[pl.ds(pbase, PC_PER)], pidx_v)
    cp = pltpu.async_copy(pc_table.at[pidx_v], prow_v, sem)
    pltpu.sync_copy(ch_idx.at[pl.ds(cbase, BH_PER)], cidx_v)

    lanes = lax.iota(jnp.int32, 16)

    def perm(k, carry):
        i16 = k * 16 + lanes
        bhg = cbase + i16
        bg = bhg // H
        hg = bhg - bg * H
        jl = ((bg >> 3) * 160 + (hg >> 1) * 16 + ((bg & 7) << 1)
              + (hg & 1)) - wid * BH_PER
        v = plsc.load_gather(cidx_v, [i16])
        plsc.store_scatter(cidx2_v, [jl], v)
        return carry

    lax.fori_loop(0, GRPS, perm, 0)
    cc = pltpu.async_copy(cluster_table.at[cidx2_v], crow_v, sem)
    cp.wait()
    pltpu.sync_copy(prow_v, pc_out.at[pl.ds(pbase, PC_PER)])
    cc.wait()

    @plsc.parallel_loop(0, BH_PER * CE // 16, unroll=4)
    def rp(i):
        j = i * 16 + lanes
        v = plsc.load_gather(crow_v, [j >> 6, j & (CE - 1)])
        plsc.store_scatter(crow2_v, [j >> 10, (j >> 7) & 7, j & 127], v)

    pltpu.sync_copy(crow2_v, cl_out.at[pl.ds(wid * (BH_PER // 16), BH_PER // 16)])


@functools.partial(
    pl.kernel,
    out_type=jax.ShapeDtypeStruct((BH // 16, 8, 128), _f32),
    mesh=_mesh,
    scratch_types=[
        pltpu.VMEM((BH_PER,), jnp.int32),
        pltpu.VMEM((BH_PER,), jnp.int32),
        pltpu.VMEM((BH_PER,), jnp.int32),
        pltpu.VMEM((BH_PER, E), _f32),
        pltpu.VMEM((BH_PER, E), _f32),
        pltpu.VMEM((OFFS * E * CE,), _f32),
        pltpu.VMEM((BH_PER, CE), _f32),
        pltpu.VMEM((BH_PER // 16, 8, 128), _f32),
        pltpu.SemaphoreType.DMA,
    ],
    compiler_params=_sc_params,
)
def _attn_ctx(st_rows, s1_rows, vwo_flat, idx0_hbm, idx1_hbm, off_hbm,
              jdst_hbm, ctx_out, idx0_v, idx1_v, off_v, s0_v, s1_v, vwo_v,
              out_v, out2_v, sem):
    wid = lax.axis_index("s") * 2 + lax.axis_index("c")
    base = wid * BH_PER
    pltpu.sync_copy(idx0_hbm.at[pl.ds(base, BH_PER)], idx0_v)
    c0 = pltpu.async_copy(st_rows.at[idx0_v], s0_v, sem)
    pltpu.sync_copy(idx1_hbm.at[pl.ds(base, BH_PER)], idx1_v)
    c1 = pltpu.async_copy(s1_rows.at[idx1_v], s1_v, sem)
    pltpu.sync_copy(off_hbm.at[pl.ds(base, BH_PER)], off_v)
    pltpu.sync_copy(vwo_flat, vwo_v)
    c0.wait()
    c1.wait()

    lanes = lax.iota(jnp.int32, 16)

    def group(g, carry):
        rows = g * 16 + lanes
        s0 = [plsc.load_gather(s0_v, [rows, jnp.full((16,), e, jnp.int32)])
              for e in range(E)]
        s1 = [plsc.load_gather(s1_v, [rows, jnp.full((16,), e, jnp.int32)])
              for e in range(E)]
        m0 = s0[0]
        m1 = s1[0]
        for e in range(1, E):
            m0 = jnp.maximum(m0, s0[e])
            m1 = jnp.maximum(m1, s1[e])
        p0 = [jnp.exp(x - m0) for x in s0]
        p1 = [jnp.exp(x - m1) for x in s1]
        z0 = p0[0]
        z1 = p1[0]
        for e in range(1, E):
            z0 = z0 + p0[e]
            z1 = z1 + p1[e]
        r0 = 0.5 / z0
        r1 = 0.5 / z1
        w = [p0[e] * r0 + p1[e] * r1 for e in range(E)]

        offv = off_v[pl.ds(g * 16, 16)]
        vbase = offv * (E * CE)
        off9 = offv * 9

        @plsc.parallel_loop(0, CE, unroll=4)
        def ctx_loop(c):
            cvec = jnp.broadcast_to(c, (16,))
            bsw = cvec + off9
            acc = jnp.zeros((16,), _f32)
            for e in range(E):
                idx = (vbase + e * CE) | ((bsw + e) & (CE - 1))
                acc = acc + w[e] * plsc.load_gather(vwo_v, [idx])
            plsc.store_scatter(out_v, [rows, (cvec + rows) & (CE - 1)], acc)

        return carry

    lax.fori_loop(0, GRPS, group, 0)

    pltpu.sync_copy(jdst_hbm.at[pl.ds(base, BH_PER)], idx0_v)

    @plsc.parallel_loop(0, BH_PER * CE // 16, unroll=4)
    def repack(i):
        j = i * 16 + lanes
        r = j >> 6
        c = j & (CE - 1)
        jl = plsc.load_gather(idx0_v, [r])
        v = plsc.load_gather(out_v, [r, (c + r) & (CE - 1)])
        n = jl * CE + c
        plsc.store_scatter(out2_v, [n >> 10, (n >> 7) & 7, n & 127], v)

    pltpu.sync_copy(out2_v, ctx_out.at[pl.ds(wid * (BH_PER // 16), BH_PER // 16)])


def _b2_body(ct_ref, wq_ref, ot2_ref, wk_ref, wv_ref, wo_ref,
             st_ref, vwo_ref, m1_ref):
    q = jnp.dot(ct_ref[...], wq_ref[...], preferred_element_type=_f32)
    k2 = jnp.dot(ot2_ref[...], wk_ref[...], preferred_element_type=_f32)
    st = lax.dot_general(q, k2, (((1,), (1,)), ((), ())),
                         preferred_element_type=_f32) * 0.125
    for t in range(4):
        st_ref[:, 8 * t:8 * t + 8, :] = \
            st[:, 128 * t:128 * t + 128].reshape(NCLUST // 64, 8, 128)

    @pl.when(pl.program_id(0) == 0)
    def _():
        v2 = jnp.dot(ot2_ref[...], wv_ref[...], preferred_element_type=_f32)
        vwo_ref[...] = jnp.dot(v2, wo_ref[...], preferred_element_type=_f32)
        m1 = lax.dot_general(wq_ref[...], k2, (((1,), (1,)), ((), ())),
                             preferred_element_type=_f32)
        m1_ref[...] = m1 * 0.125


def _b3_body(pce_ref, m1_ref, s1_ref):
    s1 = jnp.dot(pce_ref[...], m1_ref[...], preferred_element_type=_f32)
    for t in range(4):
        s1_ref[:, 8 * t:8 * t + 8, :] = \
            s1[:, 128 * t:128 * t + 128].reshape(B // 8, 8, 128)


def _d_body(pce_ref, cl_ref, ctx_ref, dpf_ref, wp_ref, wc_ref, wx_ref,
            wd_ref, bias_ref, out_ref):
    acc = jnp.dot(pce_ref[...], wp_ref[...], preferred_element_type=_f32)
    acc = acc + jnp.dot(dpf_ref[...], wd_ref[...], preferred_element_type=_f32)
    for t in range(H // 2):
        cl_t = cl_ref[:, t, :, :].reshape(B // 8, 128)
        ctx_t = ctx_ref[:, t, :, :].reshape(B // 8, 128)
        acc = acc + jnp.dot(cl_t, wc_ref[128 * t:128 * t + 128, :],
                            preferred_element_type=_f32)
        acc = acc + jnp.dot(ctx_t, wx_ref[128 * t:128 * t + 128, :],
                            preferred_element_type=_f32)
    out_ref[...] = acc + bias_ref[...]


def kernel(cluster_history, offset_history, pc, dpf_vectors, pc_table,
           cluster_table, offset_table, Wq, Wk, Wv, Wo, W_cand, b_cand,
           W_off, b_off):
    ch = cluster_history.reshape(-1)
    off = offset_history.reshape(-1)
    pc_idx = pc.reshape(-1)
    ot2 = offset_table.reshape(OFFS * E, CE)

    bh = lax.iota(jnp.int32, BH)
    bidx = bh // H
    hidx = bh % H
    jg = ((bidx >> 3) * 160 + (hidx >> 1) * 16 + ((bidx & 7) << 1) + (hidx & 1))
    jv = jg - (bidx >> 5) * BH_PER

    pc_embed, cl_embed = _gather_embeds(pc_idx, ch, pc_table, cluster_table)

    st_t, vwo, m1 = pl.pallas_call(
        _b2_body,
        grid=(8,),
        in_specs=[
            pl.BlockSpec((NCLUST // 8, CE), lambda i: (i, 0)),
            pl.BlockSpec((CE, CE), lambda i: (0, 0)),
            pl.BlockSpec((OFFS * E, CE), lambda i: (0, 0)),
            pl.BlockSpec((CE, CE), lambda i: (0, 0)),
            pl.BlockSpec((CE, CE), lambda i: (0, 0)),
            pl.BlockSpec((CE, CE), lambda i: (0, 0)),
        ],
        out_specs=[
            pl.BlockSpec((NCLUST // 64, 32, 128), lambda i: (i, 0, 0)),
            pl.BlockSpec((OFFS * E, CE), lambda i: (0, 0)),
            pl.BlockSpec((CE, OFFS * E), lambda i: (0, 0)),
        ],
        out_shape=[
            jax.ShapeDtypeStruct((NCLUST // 8, 32, 128), _f32),
            jax.ShapeDtypeStruct((OFFS * E, CE), _f32),
            jax.ShapeDtypeStruct((CE, OFFS * E), _f32),
        ],
    )(cluster_table, Wq, ot2, Wk, Wv, Wo)

    s1_t = pl.pallas_call(
        _b3_body,
        out_shape=jax.ShapeDtypeStruct((B // 8, 32, 128), _f32),
    )(pc_embed, m1)

    bidx = lax.iota(jnp.int32, BH) // H
    idx0 = ((ch >> 3) << 9) + ((off >> 4) << 7) + ((ch & 7) << 4) + (off & 15)
    idx1 = ((bidx >> 3) << 9) + ((off >> 4) << 7) + ((bidx & 7) << 4) + (off & 15)
    rr = lax.iota(jnp.int32, OFFS * E)[:, None]
    cc = lax.iota(jnp.int32, CE)[None, :]
    vwo_sw = jnp.take_along_axis(
        vwo, (cc - rr - (rr >> 3)) & (CE - 1), axis=1)
    ctx = _attn_ctx(st_t.reshape(NCLUST * OFFS, E), s1_t.reshape(B * OFFS, E),
                    vwo_sw.reshape(-1), idx0, idx1, off, jv)

    wfull = jnp.concatenate([W_cand, W_off], axis=1)
    bias = jnp.concatenate([b_cand, b_off]).reshape(1, NCAND + 1 + OFFS)
    nout = NCAND + 1 + OFFS
    out = pl.pallas_call(
        _d_body,
        grid=(8,),
        in_specs=[
            pl.BlockSpec((B // 8, PE), lambda i: (i, 0)),
            pl.BlockSpec((16, H // 2, 8, 128), lambda i: (i, 0, 0, 0)),
            pl.BlockSpec((16, H // 2, 8, 128), lambda i: (i, 0, 0, 0)),
            pl.BlockSpec((B // 8, DPFH * NCAND), lambda i: (i, 0)),
            pl.BlockSpec((PE, nout), lambda i: (0, 0)),
            pl.BlockSpec((H * CE, nout), lambda i: (0, 0)),
            pl.BlockSpec((H * CE, nout), lambda i: (0, 0)),
            pl.BlockSpec((DPFH * NCAND, nout), lambda i: (0, 0)),
            pl.BlockSpec((1, nout), lambda i: (0, 0)),
        ],
        out_specs=pl.BlockSpec((B // 8, nout), lambda i: (i, 0)),
        out_shape=jax.ShapeDtypeStruct((B, nout), _f32),
    )(pc_embed, cl_embed.reshape(B // 8, H // 2, 8, 128),
      ctx.reshape(B // 8, H // 2, 8, 128),
      dpf_vectors.reshape(B, DPFH * NCAND), wfull[:PE],
      wfull[PE:PE + H * CE], wfull[PE + H * CE:PE + 2 * H * CE],
      wfull[PE + 2 * H * CE:], bias)

    return (out[:, :NCAND + 1], out[:, NCAND + 1:])


# speedup vs baseline: 1.0544x; 1.0279x over previous
import functools
import jax
import jax.numpy as jnp
from jax import lax
from jax.experimental import pallas as pl
from jax.experimental.pallas import tpu as pltpu
from jax.experimental.pallas import tpu_sc as plsc

B = 1024
H = 20
E = 8
CE = 64
PE = 64
OFFS = 64
NCLUST = 4096
NCAND = 4
DPFH = 3
BH = B * H
NW = 32
PC_PER = B // NW
BH_PER = BH // NW
GRPS = BH_PER // 16

_mesh = plsc.VectorSubcoreMesh(core_axis_name="c", subcore_axis_name="s")
_f32 = jnp.float32
_sc_params = pltpu.CompilerParams(use_tc_tiling_on_sc=False,
                                  needs_layout_passes=False)


@functools.partial(
    pl.kernel,
    out_type=[
        jax.ShapeDtypeStruct((B, PE), _f32),
        jax.ShapeDtypeStruct((BH, CE), _f32),
    ],
    mesh=_mesh,
    scratch_types=[
        pltpu.VMEM((PC_PER,), jnp.int32),
        pltpu.VMEM((PC_PER, PE), _f32),
        pltpu.VMEM((BH_PER,), jnp.int32),
        pltpu.VMEM((BH_PER, CE), _f32),
        pltpu.SemaphoreType.DMA,
    ],
    compiler_params=_sc_params,
)
def _gather_embeds(pc_idx, ch_idx, pc_table, cluster_table,
                   pc_out, cl_out, pidx_v, prow_v, cidx_v, crow_v, sem):
    wid = lax.axis_index("s") * 2 + lax.axis_index("c")
    pbase = wid * PC_PER
    cbase = wid * BH_PER
    pltpu.sync_copy(pc_idx.at[pl.ds(pbase, PC_PER)], pidx_v)
    cp = pltpu.async_copy(pc_table.at[pidx_v], prow_v, sem)
    pltpu.sync_copy(ch_idx.at[pl.ds(cbase, BH_PER)], cidx_v)
    cc = pltpu.async_copy(cluster_table.at[cidx_v], crow_v, sem)
    cp.wait()
    pltpu.sync_copy(prow_v, pc_out.at[pl.ds(pbase, PC_PER)])
    cc.wait()
    pltpu.sync_copy(crow_v, cl_out.at[pl.ds(cbase, BH_PER)])


@functools.partial(
    pl.kernel,
    out_type=jax.ShapeDtypeStruct((BH, CE), _f32),
    mesh=_mesh,
    scratch_types=[
        pltpu.VMEM((BH_PER,), jnp.int32),
        pltpu.VMEM((BH_PER,), jnp.int32),
        pltpu.VMEM((BH_PER,), jnp.int32),
        pltpu.VMEM((BH_PER, E), _f32),
        pltpu.VMEM((BH_PER, E), _f32),
        pltpu.VMEM((OFFS * (E * (CE + 1) + 1),), _f32),
        pltpu.VMEM((BH_PER, CE + 1), _f32),
        pltpu.SemaphoreType.DMA,
    ],
    compiler_params=_sc_params,
)
def _attn_ctx(st_rows, s1_rows, vwo_flat, idx0_hbm, idx1_hbm, off_hbm,
              ctx_out, idx0_v, idx1_v, off_v, s0_v, s1_v, vwo_v,
              out_v, sem):
    wid = lax.axis_index("s") * 2 + lax.axis_index("c")
    base = wid * BH_PER
    pltpu.sync_copy(idx0_hbm.at[pl.ds(base, BH_PER)], idx0_v)
    c0 = pltpu.async_copy(st_rows.at[idx0_v], s0_v, sem)
    pltpu.sync_copy(idx1_hbm.at[pl.ds(base, BH_PER)], idx1_v)
    c1 = pltpu.async_copy(s1_rows.at[idx1_v], s1_v, sem)
    pltpu.sync_copy(off_hbm.at[pl.ds(base, BH_PER)], off_v)
    pltpu.sync_copy(vwo_flat, vwo_v)
    c0.wait()
    c1.wait()

    lanes = lax.iota(jnp.int32, 16)

    def group(g, carry):
        rows = g * 16 + lanes
        s0 = [plsc.load_gather(s0_v, [rows, jnp.full((16,), e, jnp.int32)])
              for e in range(E)]
        s1 = [plsc.load_gather(s1_v, [rows, jnp.full((16,), e, jnp.int32)])
              for e in range(E)]
        m0 = s0[0]
        m1 = s1[0]
        for e in range(1, E):
            m0 = jnp.maximum(m0, s0[e])
            m1 = jnp.maximum(m1, s1[e])
        p0 = [jnp.exp(x - m0) for x in s0]
        p1 = [jnp.exp(x - m1) for x in s1]
        z0 = p0[0]
        z1 = p1[0]
        for e in range(1, E):
            z0 = z0 + p0[e]
            z1 = z1 + p1[e]
        r0 = 0.5 / z0
        r1 = 0.5 / z1
        w = [p0[e] * r0 + p1[e] * r1 for e in range(E)]

        offv = off_v[pl.ds(g * 16, 16)]
        vbase = offv * (E * (CE + 1) + 1)

        @plsc.parallel_loop(0, CE, unroll=4)
        def ctx_loop(c):
            cvec = jnp.broadcast_to(c, (16,))
            acc = jnp.zeros((16,), _f32)
            for e in range(E):
                acc = acc + w[e] * plsc.load_gather(
                    vwo_v, [vbase + e * (CE + 1) + cvec])
            plsc.store_scatter(out_v, [rows, cvec], acc)

        return carry

    lax.fori_loop(0, GRPS, group, 0)

    pltpu.sync_copy(out_v.at[:, pl.ds(0, CE)], ctx_out.at[pl.ds(base, BH_PER)])


def _b2_body(ct_ref, wq_ref, ot2_ref, wk_ref, wv_ref, wo_ref,
             st_ref, vwo_ref, m1_ref):
    q = jnp.dot(ct_ref[...], wq_ref[...], preferred_element_type=_f32)
    k2 = jnp.dot(ot2_ref[...], wk_ref[...], preferred_element_type=_f32)
    st = lax.dot_general(q, k2, (((1,), (1,)), ((), ())),
                         preferred_element_type=_f32) * 0.125
    for t in range(4):
        st_ref[:, 8 * t:8 * t + 8, :] = \
            st[:, 128 * t:128 * t + 128].reshape(NCLUST // 64, 8, 128)

    @pl.when(pl.program_id(0) == 0)
    def _():
        v2 = jnp.dot(ot2_ref[...], wv_ref[...], preferred_element_type=_f32)
        vwo_ref[...] = jnp.dot(v2, wo_ref[...], preferred_element_type=_f32)
        m1 = lax.dot_general(wq_ref[...], k2, (((1,), (1,)), ((), ())),
                             preferred_element_type=_f32)
        m1_ref[...] = m1 * 0.125


def _b3_body(pce_ref, m1_ref, s1_ref):
    s1 = jnp.dot(pce_ref[...], m1_ref[...], preferred_element_type=_f32)
    for t in range(4):
        s1_ref[:, 8 * t:8 * t + 8, :] = \
            s1[:, 128 * t:128 * t + 128].reshape(B // 8, 8, 128)


def _d_body(pce_ref, cl_ref, ctx_ref, dpf_ref, wp_ref, wc_ref, wx_ref,
            wd_ref, bias_ref, out_ref):
    acc = jnp.dot(pce_ref[...], wp_ref[...], preferred_element_type=_f32)
    acc = acc + jnp.dot(cl_ref[...], wc_ref[...], preferred_element_type=_f32)
    acc = acc + jnp.dot(ctx_ref[...], wx_ref[...], preferred_element_type=_f32)
    acc = acc + jnp.dot(dpf_ref[...], wd_ref[...], preferred_element_type=_f32)
    out_ref[...] = acc + bias_ref[...]


def kernel(cluster_history, offset_history, pc, dpf_vectors, pc_table,
           cluster_table, offset_table, Wq, Wk, Wv, Wo, W_cand, b_cand,
           W_off, b_off):
    ch = cluster_history.reshape(-1)
    off = offset_history.reshape(-1)
    pc_idx = pc.reshape(-1)
    ot2 = offset_table.reshape(OFFS * E, CE)

    pc_embed, cl_embed = _gather_embeds(pc_idx, ch, pc_table, cluster_table)

    st_t, vwo, m1 = pl.pallas_call(
        _b2_body,
        grid=(8,),
        in_specs=[
            pl.BlockSpec((NCLUST // 8, CE), lambda i: (i, 0)),
            pl.BlockSpec((CE, CE), lambda i: (0, 0)),
            pl.BlockSpec((OFFS * E, CE), lambda i: (0, 0)),
            pl.BlockSpec((CE, CE), lambda i: (0, 0)),
            pl.BlockSpec((CE, CE), lambda i: (0, 0)),
            pl.BlockSpec((CE, CE), lambda i: (0, 0)),
        ],
        out_specs=[
            pl.BlockSpec((NCLUST // 64, 32, 128), lambda i: (i, 0, 0)),
            pl.BlockSpec((OFFS * E, CE), lambda i: (0, 0)),
            pl.BlockSpec((CE, OFFS * E), lambda i: (0, 0)),
        ],
        out_shape=[
            jax.ShapeDtypeStruct((NCLUST // 8, 32, 128), _f32),
            jax.ShapeDtypeStruct((OFFS * E, CE), _f32),
            jax.ShapeDtypeStruct((CE, OFFS * E), _f32),
        ],
    )(cluster_table, Wq, ot2, Wk, Wv, Wo)

    s1_t = pl.pallas_call(
        _b3_body,
        out_shape=jax.ShapeDtypeStruct((B // 8, 32, 128), _f32),
    )(pc_embed, m1)

    bidx = lax.iota(jnp.int32, BH) // H
    idx0 = ((ch >> 3) << 9) + ((off >> 4) << 7) + ((ch & 7) << 4) + (off & 15)
    idx1 = ((bidx >> 3) << 9) + ((off >> 4) << 7) + ((bidx & 7) << 4) + (off & 15)
    vwo_pad = jnp.pad(vwo, ((0, 0), (0, 1))).reshape(OFFS, E * (CE + 1))
    vwo_pad = jnp.pad(vwo_pad, ((0, 0), (0, 1)))
    ctx = _attn_ctx(st_t.reshape(NCLUST * OFFS, E), s1_t.reshape(B * OFFS, E),
                    vwo_pad.reshape(-1), idx0, idx1, off)

    wfull = jnp.concatenate([W_cand, W_off], axis=1)
    bias = jnp.concatenate([b_cand, b_off]).reshape(1, NCAND + 1 + OFFS)
    nout = NCAND + 1 + OFFS
    out = pl.pallas_call(
        _d_body,
        grid=(8,),
        in_specs=[
            pl.BlockSpec((B // 8, PE), lambda i: (i, 0)),
            pl.BlockSpec((B // 8, H * CE), lambda i: (i, 0)),
            pl.BlockSpec((B // 8, H * CE), lambda i: (i, 0)),
            pl.BlockSpec((B // 8, DPFH * NCAND), lambda i: (i, 0)),
            pl.BlockSpec((PE, nout), lambda i: (0, 0)),
            pl.BlockSpec((H * CE, nout), lambda i: (0, 0)),
            pl.BlockSpec((H * CE, nout), lambda i: (0, 0)),
            pl.BlockSpec((DPFH * NCAND, nout), lambda i: (0, 0)),
            pl.BlockSpec((1, nout), lambda i: (0, 0)),
        ],
        out_specs=pl.BlockSpec((B // 8, nout), lambda i: (i, 0)),
        out_shape=jax.ShapeDtypeStruct((B, nout), _f32),
    )(pc_embed, cl_embed.reshape(B, H * CE), ctx.reshape(B, H * CE),
      dpf_vectors.reshape(B, DPFH * NCAND), wfull[:PE],
      wfull[PE:PE + H * CE], wfull[PE + H * CE:PE + 2 * H * CE],
      wfull[PE + 2 * H * CE:], bias)

    return (out[:, :NCAND + 1], out[:, NCAND + 1:])
